# baseline (speedup 1.0000x reference)
import jax
import jax.numpy as jnp
from jax.experimental import pallas as pl
from jax.experimental.pallas import tpu as pltpu
from jax.experimental.pallas import tpu_sc as plsc

NPAD = 10240


def _dott(a, b):
    return jax.lax.dot_general(a, b, (((1,), (1,)), ((), ())),
                               preferred_element_type=jnp.float32)
W = 125
WC = 10000
N_WORKERS = 32


def _counts_sc(node_type, src2, dst2, z1d):
    nw2 = src2.shape[0]
    nw_per = nw2 // N_WORKERS
    rps = 4 * NPAD // 16
    mesh = plsc.VectorSubcoreMesh(core_axis_name="c", subcore_axis_name="s")

    @pl.kernel(
        out_type=jax.ShapeDtypeStruct((2, 4 * NPAD), jnp.float32),
        mesh=mesh,
        scratch_types=[
            pltpu.VMEM((WC,), jnp.int32),
            pltpu.VMEM((WC,), jnp.int32),
            pltpu.VMEM((WC,), jnp.int32),
            pltpu.VMEM((WC,), jnp.int32),
            pltpu.VMEM((WC,), jnp.float32),
            pltpu.VMEM_SHARED((4 * NPAD,), jnp.float32),
            pltpu.SemaphoreType.DMA,
        ],
    )
    def k(nt_hbm, src_hbm, dst_hbm, z_hbm, out_hbm,
          src_v, dst_v, t_v, idx_v, ones_v, acc, sem):
        cid = jax.lax.axis_index("c")
        sid = jax.lax.axis_index("s")
        wid = cid * 16 + sid
        pltpu.sync_copy(z_hbm, acc.at[pl.ds(sid * rps, rps)])

        @pl.loop(0, WC, step=16)
        def _(i):
            ones_v.at[pl.ds(i, 16)][...] = jnp.full((16,), 1.0, jnp.float32)

        plsc.subcore_barrier()

        pltpu.sync_copy(src_hbm.at[wid], src_v)
        pltpu.sync_copy(dst_hbm.at[wid], dst_v)
        pltpu.async_copy(nt_hbm.at[src_v], t_v, sem).wait()

        @pl.loop(0, WC, step=16)
        def _(i):
            s = pl.ds(i, 16)
            idx_v.at[s][...] = dst_v.at[s][...] * 4 + t_v.at[s][...]

        pltpu.sync_copy(ones_v, acc.at[idx_v], add=True)

        plsc.subcore_barrier()
        pltpu.sync_copy(
            acc.at[pl.ds(sid * rps, rps)], out_hbm.at[cid].at[pl.ds(sid * rps, rps)]
        )

    return k(node_type, src2, dst2, z1d)


def _segsum_sc(x, src3, dst3, zeros):
    f = x.shape[1]
    nw = src3.shape[0]
    nw_per = nw // N_WORKERS
    rps = NPAD // 16
    mesh = plsc.VectorSubcoreMesh(core_axis_name="c", subcore_axis_name="s")

    @pl.kernel(
        out_type=jax.ShapeDtypeStruct((2, NPAD, f), jnp.float32),
        mesh=mesh,
        scratch_types=(
            [pltpu.VMEM((1, W), jnp.int32)] * 8
            + [
                pltpu.VMEM((W, f), jnp.float32),
                pltpu.VMEM((W, f), jnp.float32),
                pltpu.VMEM_SHARED((NPAD, f), jnp.float32),
                pltpu.SemaphoreType.DMA,
                pltpu.SemaphoreType.DMA,
                pltpu.SemaphoreType.DMA,
            ]
        ),
    )
    def k(x_hbm, src_hbm, dst_hbm, z_hbm, out_hbm,
          sa0, da0, sb0, db0, sa1, da1, sb1, db1,
          rows0, rows1, acc, sem0, sem1, semi):
        cid = jax.lax.axis_index("c")
        sid = jax.lax.axis_index("s")
        wid = cid * 16 + sid
        j0 = wid * nw_per
        pltpu.sync_copy(z_hbm, acc.at[pl.ds(sid * rps, rps)])
        plsc.subcore_barrier()

        def phase(rows, sem, d_act, s_pre, d_pre, j_new):
            hs = pltpu.async_copy(src_hbm.at[j_new], s_pre, semi)
            hd = pltpu.async_copy(dst_hbm.at[j_new], d_pre, semi)
            pltpu.make_async_copy(x_hbm.at[s_pre.at[0]], rows, sem).wait()
            pltpu.sync_copy(rows, acc.at[d_act.at[0]], add=True)
            hs.wait()
            hd.wait()
            pltpu.async_copy(x_hbm.at[s_pre.at[0]], rows, sem)

        pltpu.sync_copy(src_hbm.at[j0], sa0)
        pltpu.sync_copy(dst_hbm.at[j0], da0)
        pltpu.async_copy(x_hbm.at[sa0.at[0]], rows0, sem0)
        pltpu.sync_copy(src_hbm.at[j0 + 1], sa1)
        pltpu.sync_copy(dst_hbm.at[j0 + 1], da1)
        pltpu.async_copy(x_hbm.at[sa1.at[0]], rows1, sem1)

        @pl.loop(0, nw_per - 4, step=4)
        def _(w):
            j = j0 + w
            phase(rows0, sem0, da0, sb0, db0, j + 2)
            phase(rows1, sem1, da1, sb1, db1, j + 3)
            phase(rows0, sem0, db0, sa0, da0, j + 4)
            phase(rows1, sem1, db1, sa1, da1, j + 5)

        phase(rows0, sem0, da0, sb0, db0, j0 + nw_per - 2)
        phase(rows1, sem1, da1, sb1, db1, j0 + nw_per - 1)
        pltpu.make_async_copy(x_hbm.at[sb0.at[0]], rows0, sem0).wait()
        pltpu.sync_copy(rows0, acc.at[db0.at[0]], add=True)
        pltpu.make_async_copy(x_hbm.at[sb1.at[0]], rows1, sem1).wait()
        pltpu.sync_copy(rows1, acc.at[db1.at[0]], add=True)

        plsc.subcore_barrier()
        pltpu.sync_copy(
            acc.at[pl.ds(sid * rps, rps)], out_hbm.at[cid].at[pl.ds(sid * rps, rps)]
        )

    return k(x, src3, dst3, zeros)


def _layer1_tc(c0, c1, nt2, node_table, wl1t, bl1, wr1t, n, d):
    bn = 2000

    def body(c0_ref, c1_ref, nt_ref, tab_ref, wl_ref, bl_ref, wr_ref, o_ref):
        c4 = c0_ref[...] + c1_ref[...]
        deg = jnp.sum(c4, axis=1, keepdims=True)
        inv = 1.0 / jnp.maximum(deg, 1.0)
        io = jax.lax.broadcasted_iota(jnp.int32, (1, 4), 1)
        oh4 = (nt_ref[...] == io).astype(jnp.float32)
        p = _dott(tab_ref[...], wl_ref[...])
        q = _dott(tab_ref[...], wr_ref[...])
        agg = jnp.dot(c4, p, preferred_element_type=jnp.float32) * inv
        self_t = jnp.dot(oh4, q, preferred_element_type=jnp.float32)
        o_ref[...] = jnp.maximum(agg + bl_ref[...] + self_t, 0.0)

    return pl.pallas_call(
        body,
        grid=(n // bn,),
        in_specs=[
            pl.BlockSpec((bn, 4), lambda i: (i, 0)),
            pl.BlockSpec((bn, 4), lambda i: (i, 0)),
            pl.BlockSpec((bn, 1), lambda i: (i, 0)),
            pl.BlockSpec((4, d), lambda i: (0, 0)),
            pl.BlockSpec((d, d), lambda i: (0, 0)),
            pl.BlockSpec((1, d), lambda i: (0, 0)),
            pl.BlockSpec((d, d), lambda i: (0, 0)),
        ],
        out_specs=pl.BlockSpec((bn, d), lambda i: (i, 0)),
        out_shape=jax.ShapeDtypeStruct((n, d), jnp.float32),
    )(c0, c1, nt2, node_table, wl1t, bl1.reshape(1, d), wr1t)


def _layer2_head_tc(p0, p1, c0, c1, x1, wl2t, bl2, wr2t, h1t, hb1, h2t, hb2, n, d, c):
    bn = 2000

    def body(p0_ref, p1_ref, c0_ref, c1_ref, x1_ref, wl_ref, bl_ref, wr_ref,
             h1_ref, hb1_ref, h2_ref, hb2_ref, o_ref):
        c4 = c0_ref[...] + c1_ref[...]
        deg = jnp.sum(c4, axis=1, keepdims=True)
        inv = 1.0 / jnp.maximum(deg, 1.0)
        agg = (p0_ref[...] + p1_ref[...]) * inv
        x1b = x1_ref[...]
        x2 = jnp.maximum(
            _dott(agg, wl_ref[...]) + bl_ref[...] + _dott(x1b, wr_ref[...]),
            0.0,
        )
        h = jnp.maximum(_dott(x2, h1_ref[...]) + hb1_ref[...], 0.0)
        o_ref[...] = _dott(h, h2_ref[...]) + hb2_ref[...]

    return pl.pallas_call(
        body,
        grid=(n // bn,),
        in_specs=[
            pl.BlockSpec((bn, d), lambda i: (i, 0)),
            pl.BlockSpec((bn, d), lambda i: (i, 0)),
            pl.BlockSpec((bn, 4), lambda i: (i, 0)),
            pl.BlockSpec((bn, 4), lambda i: (i, 0)),
            pl.BlockSpec((bn, d), lambda i: (i, 0)),
            pl.BlockSpec((d, d), lambda i: (0, 0)),
            pl.BlockSpec((1, d), lambda i: (0, 0)),
            pl.BlockSpec((d, d), lambda i: (0, 0)),
            pl.BlockSpec((d, d), lambda i: (0, 0)),
            pl.BlockSpec((1, d), lambda i: (0, 0)),
            pl.BlockSpec((c, d), lambda i: (0, 0)),
            pl.BlockSpec((1, c), lambda i: (0, 0)),
        ],
        out_specs=pl.BlockSpec((bn, c), lambda i: (i, 0)),
        out_shape=jax.ShapeDtypeStruct((n, c), jnp.float32),
    )(p0, p1, c0, c1, x1, wl2t, bl2.reshape(1, d), wr2t, h1t,
      hb1.reshape(1, d), h2t, hb2.reshape(1, c))


def kernel(node_type, edge_type, edge_index, y, node_table, edge_table,
           Wl1, bl1, Wr1, Wl2, bl2, Wr2, H1, hb1, H2, hb2):
    n = node_type.shape[0]
    e = edge_index.shape[1]
    d = node_table.shape[1]
    c = H2.shape[0]
    nw = e // W
    nw2 = e // WC

    src3 = edge_index[0].reshape(nw, 1, W)
    dst3 = edge_index[1].reshape(nw, 1, W)
    src2 = edge_index[0].reshape(nw2, WC)
    dst2 = edge_index[1].reshape(nw2, WC)
    z1d = jnp.zeros((4 * NPAD // 16,), jnp.float32)
    z128 = jnp.zeros((NPAD // 16, d), jnp.float32)

    cnt = _counts_sc(node_type, src2, dst2, z1d)
    c0 = cnt[0, :4 * n].reshape(n, 4)
    c1 = cnt[1, :4 * n].reshape(n, 4)
    nt2 = node_type.reshape(n, 1)
    x1 = _layer1_tc(c0, c1, nt2, node_table, Wl1, bl1, Wr1, n, d)
    agg = _segsum_sc(x1, src3, dst3, z128)
    pred = _layer2_head_tc(agg[0, :n], agg[1, :n], c0, c1, x1,
                           Wl2, bl2, Wr2, H1, hb1, H2, hb2, n, d, c)
    true_class = y[:, 1].astype(jnp.int32)
    return (pred, true_class, y)


# speedup vs baseline: 1.0197x; 1.0197x over previous
import jax
import jax.numpy as jnp
from jax.experimental import pallas as pl
from jax.experimental.pallas import tpu as pltpu
from jax.experimental.pallas import tpu_sc as plsc

NPAD = 10240


def _dott(a, b):
    return jax.lax.dot_general(a, b, (((1,), (1,)), ((), ())),
                               preferred_element_type=jnp.float32)
W = 125
WC = 2000
N_WORKERS = 32


def _counts_sc(node_type, src2, dst2, z1d):
    nw2 = src2.shape[0]
    nw_per = nw2 // N_WORKERS
    rps = 4 * NPAD // 16
    mesh = plsc.VectorSubcoreMesh(core_axis_name="c", subcore_axis_name="s")

    @pl.kernel(
        out_type=jax.ShapeDtypeStruct((2, 4 * NPAD), jnp.float32),
        mesh=mesh,
        scratch_types=(
            [pltpu.VMEM((WC,), jnp.int32)] * 8
            + [
                pltpu.VMEM((WC,), jnp.float32),
                pltpu.VMEM_SHARED((4 * NPAD,), jnp.float32),
                pltpu.SemaphoreType.DMA,
                pltpu.SemaphoreType.DMA,
            ]
        ),
    )
    def k(nt_hbm, src_hbm, dst_hbm, z_hbm, out_hbm,
          src_a, dst_a, t_a, idx_a, src_b, dst_b, t_b, idx_b,
          ones_v, acc, sem_a, sem_b):
        cid = jax.lax.axis_index("c")
        sid = jax.lax.axis_index("s")
        wid = cid * 16 + sid
        pltpu.sync_copy(z_hbm, acc.at[pl.ds(sid * rps, rps)])

        @pl.loop(0, WC, step=16)
        def _(i):
            ones_v.at[pl.ds(i, 16)][...] = jnp.full((16,), 1.0, jnp.float32)

        plsc.subcore_barrier()

        bufs = [(src_a, dst_a, t_a, idx_a, sem_a),
                (src_b, dst_b, t_b, idx_b, sem_b)]
        j0 = wid * nw_per
        pltpu.sync_copy(src_hbm.at[j0], src_a)
        pltpu.sync_copy(dst_hbm.at[j0], dst_a)
        pltpu.async_copy(nt_hbm.at[src_a], t_a, sem_a)
        for w in range(nw_per):
            sv, dv, tv, iv, sm = bufs[w % 2]
            if w + 1 < nw_per:
                nsv, ndv, ntv, _, nsm = bufs[(w + 1) % 2]
                pltpu.sync_copy(src_hbm.at[j0 + w + 1], nsv)
                pltpu.sync_copy(dst_hbm.at[j0 + w + 1], ndv)
                pltpu.async_copy(nt_hbm.at[nsv], ntv, nsm)
            pltpu.make_async_copy(nt_hbm.at[sv], tv, sm).wait()

            @pl.loop(0, WC, step=16)
            def _(i):
                s = pl.ds(i, 16)
                iv.at[s][...] = dv.at[s][...] * 4 + tv.at[s][...]

            pltpu.sync_copy(ones_v, acc.at[iv], add=True)

        plsc.subcore_barrier()
        pltpu.sync_copy(
            acc.at[pl.ds(sid * rps, rps)], out_hbm.at[cid].at[pl.ds(sid * rps, rps)]
        )

    return k(node_type, src2, dst2, z1d)


def _segsum_sc(x, src3, dst3, zeros):
    f = x.shape[1]
    nw = src3.shape[0]
    nw_per = nw // N_WORKERS
    rps = NPAD // 16
    mesh = plsc.VectorSubcoreMesh(core_axis_name="c", subcore_axis_name="s")

    @pl.kernel(
        out_type=jax.ShapeDtypeStruct((2, NPAD, f), jnp.float32),
        mesh=mesh,
        scratch_types=(
            [pltpu.VMEM((1, W), jnp.int32)] * 8
            + [
                pltpu.VMEM((W, f), jnp.float32),
                pltpu.VMEM((W, f), jnp.float32),
                pltpu.VMEM_SHARED((NPAD, f), jnp.float32),
                pltpu.SemaphoreType.DMA,
                pltpu.SemaphoreType.DMA,
                pltpu.SemaphoreType.DMA,
            ]
        ),
    )
    def k(x_hbm, src_hbm, dst_hbm, z_hbm, out_hbm,
          sa0, da0, sb0, db0, sa1, da1, sb1, db1,
          rows0, rows1, acc, sem0, sem1, semi):
        cid = jax.lax.axis_index("c")
        sid = jax.lax.axis_index("s")
        wid = cid * 16 + sid
        j0 = wid * nw_per
        pltpu.sync_copy(z_hbm, acc.at[pl.ds(sid * rps, rps)])
        plsc.subcore_barrier()

        def phase(rows, sem, d_act, s_pre, d_pre, j_new):
            hs = pltpu.async_copy(src_hbm.at[j_new], s_pre, semi)
            hd = pltpu.async_copy(dst_hbm.at[j_new], d_pre, semi)
            pltpu.make_async_copy(x_hbm.at[s_pre.at[0]], rows, sem).wait()
            pltpu.sync_copy(rows, acc.at[d_act.at[0]], add=True)
            hs.wait()
            hd.wait()
            pltpu.async_copy(x_hbm.at[s_pre.at[0]], rows, sem)

        pltpu.sync_copy(src_hbm.at[j0], sa0)
        pltpu.sync_copy(dst_hbm.at[j0], da0)
        pltpu.async_copy(x_hbm.at[sa0.at[0]], rows0, sem0)
        pltpu.sync_copy(src_hbm.at[j0 + 1], sa1)
        pltpu.sync_copy(dst_hbm.at[j0 + 1], da1)
        pltpu.async_copy(x_hbm.at[sa1.at[0]], rows1, sem1)

        @pl.loop(0, nw_per - 4, step=4)
        def _(w):
            j = j0 + w
            phase(rows0, sem0, da0, sb0, db0, j + 2)
            phase(rows1, sem1, da1, sb1, db1, j + 3)
            phase(rows0, sem0, db0, sa0, da0, j + 4)
            phase(rows1, sem1, db1, sa1, da1, j + 5)

        phase(rows0, sem0, da0, sb0, db0, j0 + nw_per - 2)
        phase(rows1, sem1, da1, sb1, db1, j0 + nw_per - 1)
        pltpu.make_async_copy(x_hbm.at[sb0.at[0]], rows0, sem0).wait()
        pltpu.sync_copy(rows0, acc.at[db0.at[0]], add=True)
        pltpu.make_async_copy(x_hbm.at[sb1.at[0]], rows1, sem1).wait()
        pltpu.sync_copy(rows1, acc.at[db1.at[0]], add=True)

        plsc.subcore_barrier()
        pltpu.sync_copy(
            acc.at[pl.ds(sid * rps, rps)], out_hbm.at[cid].at[pl.ds(sid * rps, rps)]
        )

    return k(x, src3, dst3, zeros)


def _layer1_tc(c0, c1, nt2, node_table, wl1t, bl1, wr1t, n, d):
    bn = 2000

    def body(c0_ref, c1_ref, nt_ref, tab_ref, wl_ref, bl_ref, wr_ref, o_ref):
        c4 = c0_ref[...] + c1_ref[...]
        deg = jnp.sum(c4, axis=1, keepdims=True)
        inv = 1.0 / jnp.maximum(deg, 1.0)
        io = jax.lax.broadcasted_iota(jnp.int32, (1, 4), 1)
        oh4 = (nt_ref[...] == io).astype(jnp.float32)
        p = _dott(tab_ref[...], wl_ref[...])
        q = _dott(tab_ref[...], wr_ref[...])
        agg = jnp.dot(c4, p, preferred_element_type=jnp.float32) * inv
        self_t = jnp.dot(oh4, q, preferred_element_type=jnp.float32)
        o_ref[...] = jnp.maximum(agg + bl_ref[...] + self_t, 0.0)

    return pl.pallas_call(
        body,
        grid=(n // bn,),
        in_specs=[
            pl.BlockSpec((bn, 4), lambda i: (i, 0)),
            pl.BlockSpec((bn, 4), lambda i: (i, 0)),
            pl.BlockSpec((bn, 1), lambda i: (i, 0)),
            pl.BlockSpec((4, d), lambda i: (0, 0)),
            pl.BlockSpec((d, d), lambda i: (0, 0)),
            pl.BlockSpec((1, d), lambda i: (0, 0)),
            pl.BlockSpec((d, d), lambda i: (0, 0)),
        ],
        out_specs=pl.BlockSpec((bn, d), lambda i: (i, 0)),
        out_shape=jax.ShapeDtypeStruct((n, d), jnp.float32),
    )(c0, c1, nt2, node_table, wl1t, bl1.reshape(1, d), wr1t)


def _layer2_head_tc(p0, p1, c0, c1, x1, wl2t, bl2, wr2t, h1t, hb1, h2t, hb2, n, d, c):
    bn = 2000

    def body(p0_ref, p1_ref, c0_ref, c1_ref, x1_ref, wl_ref, bl_ref, wr_ref,
             h1_ref, hb1_ref, h2_ref, hb2_ref, o_ref):
        c4 = c0_ref[...] + c1_ref[...]
        deg = jnp.sum(c4, axis=1, keepdims=True)
        inv = 1.0 / jnp.maximum(deg, 1.0)
        agg = (p0_ref[...] + p1_ref[...]) * inv
        x1b = x1_ref[...]
        x2 = jnp.maximum(
            _dott(agg, wl_ref[...]) + bl_ref[...] + _dott(x1b, wr_ref[...]),
            0.0,
        )
        h = jnp.maximum(_dott(x2, h1_ref[...]) + hb1_ref[...], 0.0)
        o_ref[...] = _dott(h, h2_ref[...]) + hb2_ref[...]

    return pl.pallas_call(
        body,
        grid=(n // bn,),
        in_specs=[
            pl.BlockSpec((bn, d), lambda i: (i, 0)),
            pl.BlockSpec((bn, d), lambda i: (i, 0)),
            pl.BlockSpec((bn, 4), lambda i: (i, 0)),
            pl.BlockSpec((bn, 4), lambda i: (i, 0)),
            pl.BlockSpec((bn, d), lambda i: (i, 0)),
            pl.BlockSpec((d, d), lambda i: (0, 0)),
            pl.BlockSpec((1, d), lambda i: (0, 0)),
            pl.BlockSpec((d, d), lambda i: (0, 0)),
            pl.BlockSpec((d, d), lambda i: (0, 0)),
            pl.BlockSpec((1, d), lambda i: (0, 0)),
            pl.BlockSpec((c, d), lambda i: (0, 0)),
            pl.BlockSpec((1, c), lambda i: (0, 0)),
        ],
        out_specs=pl.BlockSpec((bn, c), lambda i: (i, 0)),
        out_shape=jax.ShapeDtypeStruct((n, c), jnp.float32),
    )(p0, p1, c0, c1, x1, wl2t, bl2.reshape(1, d), wr2t, h1t,
      hb1.reshape(1, d), h2t, hb2.reshape(1, c))


def kernel(node_type, edge_type, edge_index, y, node_table, edge_table,
           Wl1, bl1, Wr1, Wl2, bl2, Wr2, H1, hb1, H2, hb2):
    n = node_type.shape[0]
    e = edge_index.shape[1]
    d = node_table.shape[1]
    c = H2.shape[0]
    nw = e // W
    nw2 = e // WC

    src3 = edge_index[0].reshape(nw, 1, W)
    dst3 = edge_index[1].reshape(nw, 1, W)
    src2 = edge_index[0].reshape(nw2, WC)
    dst2 = edge_index[1].reshape(nw2, WC)
    z1d = jnp.zeros((4 * NPAD // 16,), jnp.float32)
    z128 = jnp.zeros((NPAD // 16, d), jnp.float32)

    cnt = _counts_sc(node_type, src2, dst2, z1d)
    c0 = cnt[0, :4 * n].reshape(n, 4)
    c1 = cnt[1, :4 * n].reshape(n, 4)
    nt2 = node_type.reshape(n, 1)
    x1 = _layer1_tc(c0, c1, nt2, node_table, Wl1, bl1, Wr1, n, d)
    agg = _segsum_sc(x1, src3, dst3, z128)
    pred = _layer2_head_tc(agg[0, :n], agg[1, :n], c0, c1, x1,
                           Wl2, bl2, Wr2, H1, hb1, H2, hb2, n, d, c)
    true_class = y[:, 1].astype(jnp.int32)
    return (pred, true_class, y)


# speedup vs baseline: 1.0612x; 1.0407x over previous
import jax
import jax.numpy as jnp
from jax.experimental import pallas as pl
from jax.experimental.pallas import tpu as pltpu
from jax.experimental.pallas import tpu_sc as plsc

NPAD = 10240


def _dott(a, b):
    return jax.lax.dot_general(a, b, (((1,), (1,)), ((), ())),
                               preferred_element_type=jnp.float32)
W = 125
WC = 2000
N_WORKERS = 32


def _counts_sc(node_type, src2, dst2, z1d):
    nw2 = src2.shape[0]
    nw_per = nw2 // N_WORKERS
    rps = 4 * NPAD // 16
    mesh = plsc.VectorSubcoreMesh(core_axis_name="c", subcore_axis_name="s")

    @pl.kernel(
        out_type=jax.ShapeDtypeStruct((2, 4 * NPAD), jnp.float32),
        mesh=mesh,
        scratch_types=(
            [pltpu.VMEM((WC,), jnp.int32)] * 8
            + [
                pltpu.VMEM((WC,), jnp.float32),
                pltpu.VMEM_SHARED((4 * NPAD,), jnp.float32),
                pltpu.SemaphoreType.DMA,
                pltpu.SemaphoreType.DMA,
            ]
        ),
    )
    def k(nt_hbm, src_hbm, dst_hbm, z_hbm, out_hbm,
          src_a, dst_a, t_a, idx_a, src_b, dst_b, t_b, idx_b,
          ones_v, acc, sem_a, sem_b):
        cid = jax.lax.axis_index("c")
        sid = jax.lax.axis_index("s")
        wid = cid * 16 + sid
        pltpu.sync_copy(z_hbm, acc.at[pl.ds(sid * rps, rps)])

        @pl.loop(0, WC, step=16)
        def _(i):
            ones_v.at[pl.ds(i, 16)][...] = jnp.full((16,), 1.0, jnp.float32)

        plsc.subcore_barrier()

        bufs = [(src_a, dst_a, t_a, idx_a, sem_a),
                (src_b, dst_b, t_b, idx_b, sem_b)]
        j0 = wid * nw_per
        pltpu.sync_copy(src_hbm.at[j0], src_a)
        pltpu.sync_copy(dst_hbm.at[j0], dst_a)
        pltpu.async_copy(nt_hbm.at[src_a], t_a, sem_a)
        for w in range(nw_per):
            sv, dv, tv, iv, sm = bufs[w % 2]
            if w + 1 < nw_per:
                nsv, ndv, ntv, _, nsm = bufs[(w + 1) % 2]
                pltpu.sync_copy(src_hbm.at[j0 + w + 1], nsv)
                pltpu.sync_copy(dst_hbm.at[j0 + w + 1], ndv)
                pltpu.async_copy(nt_hbm.at[nsv], ntv, nsm)
            pltpu.make_async_copy(nt_hbm.at[sv], tv, sm).wait()

            @pl.loop(0, WC, step=16)
            def _(i):
                s = pl.ds(i, 16)
                iv.at[s][...] = dv.at[s][...] * 4 + tv.at[s][...]

            pltpu.sync_copy(ones_v, acc.at[iv], add=True)

        plsc.subcore_barrier()
        pltpu.sync_copy(
            acc.at[pl.ds(sid * rps, rps)], out_hbm.at[cid].at[pl.ds(sid * rps, rps)]
        )

    return k(node_type, src2, dst2, z1d)


def _segsum_sc(x, src3, dst3, zeros):
    f = x.shape[1]
    nw = src3.shape[0]
    nw_per = nw // N_WORKERS
    rps = NPAD // 16
    mesh = plsc.VectorSubcoreMesh(core_axis_name="c", subcore_axis_name="s")

    @pl.kernel(
        out_type=jax.ShapeDtypeStruct((2, NPAD, f), jnp.float32),
        mesh=mesh,
        scratch_types=(
            [pltpu.VMEM((1, W), jnp.int32)] * 8
            + [
                pltpu.VMEM((W, f), jnp.float32),
                pltpu.VMEM((W, f), jnp.float32),
                pltpu.VMEM_SHARED((NPAD, f), jnp.float32),
                pltpu.SemaphoreType.DMA,
                pltpu.SemaphoreType.DMA,
                pltpu.SemaphoreType.DMA,
            ]
        ),
    )
    def k(x_hbm, src_hbm, dst_hbm, z_hbm, out_hbm,
          sa0, da0, sb0, db0, sa1, da1, sb1, db1,
          rows0, rows1, acc, sem0, sem1, semi):
        cid = jax.lax.axis_index("c")
        sid = jax.lax.axis_index("s")
        wid = cid * 16 + sid
        j0 = wid * nw_per
        pltpu.sync_copy(z_hbm, acc.at[pl.ds(sid * rps, rps)])
        plsc.subcore_barrier()

        def phase(rows, sem, d_act, s_pre, d_pre, j_new):
            hs = pltpu.async_copy(src_hbm.at[j_new], s_pre, semi)
            hd = pltpu.async_copy(dst_hbm.at[j_new], d_pre, semi)
            pltpu.make_async_copy(x_hbm.at[s_pre.at[0]], rows, sem).wait()
            pltpu.sync_copy(rows, acc.at[d_act.at[0]], add=True)
            hs.wait()
            hd.wait()
            pltpu.async_copy(x_hbm.at[s_pre.at[0]], rows, sem)

        pltpu.sync_copy(src_hbm.at[j0], sa0)
        pltpu.sync_copy(dst_hbm.at[j0], da0)
        pltpu.async_copy(x_hbm.at[sa0.at[0]], rows0, sem0)
        pltpu.sync_copy(src_hbm.at[j0 + 1], sa1)
        pltpu.sync_copy(dst_hbm.at[j0 + 1], da1)
        pltpu.async_copy(x_hbm.at[sa1.at[0]], rows1, sem1)

        @pl.loop(0, nw_per - 4, step=4)
        def _(w):
            j = j0 + w
            phase(rows0, sem0, da0, sb0, db0, j + 2)
            phase(rows1, sem1, da1, sb1, db1, j + 3)
            phase(rows0, sem0, db0, sa0, da0, j + 4)
            phase(rows1, sem1, db1, sa1, da1, j + 5)

        phase(rows0, sem0, da0, sb0, db0, j0 + nw_per - 2)
        phase(rows1, sem1, da1, sb1, db1, j0 + nw_per - 1)
        pltpu.make_async_copy(x_hbm.at[sb0.at[0]], rows0, sem0).wait()
        pltpu.sync_copy(rows0, acc.at[db0.at[0]], add=True)
        pltpu.make_async_copy(x_hbm.at[sb1.at[0]], rows1, sem1).wait()
        pltpu.sync_copy(rows1, acc.at[db1.at[0]], add=True)

        plsc.subcore_barrier()
        pltpu.sync_copy(
            acc.at[pl.ds(sid * rps, rps)], out_hbm.at[cid].at[pl.ds(sid * rps, rps)]
        )

    return k(x, src3, dst3, zeros)


def _layer1_tc(cnt3, nt2, node_table, wl1t, bl1, wr1t, n, d):
    bn = 2000

    def body(c0_ref, c1_ref, nt_ref, tab_ref, wl_ref, bl_ref, wr_ref, o_ref):
        c4 = c0_ref[0] + c1_ref[0]
        deg = jnp.sum(c4, axis=1, keepdims=True)
        inv = 1.0 / jnp.maximum(deg, 1.0)
        io = jax.lax.broadcasted_iota(jnp.int32, (1, 4), 1)
        oh4 = (nt_ref[...] == io).astype(jnp.float32)
        p = _dott(tab_ref[...], wl_ref[...])
        q = _dott(tab_ref[...], wr_ref[...])
        agg = jnp.dot(c4, p, preferred_element_type=jnp.float32) * inv
        self_t = jnp.dot(oh4, q, preferred_element_type=jnp.float32)
        o_ref[...] = jnp.maximum(agg + bl_ref[...] + self_t, 0.0)

    return pl.pallas_call(
        body,
        grid=(n // bn,),
        in_specs=[
            pl.BlockSpec((1, bn, 4), lambda i: (0, i, 0)),
            pl.BlockSpec((1, bn, 4), lambda i: (1, i, 0)),
            pl.BlockSpec((bn, 1), lambda i: (i, 0)),
            pl.BlockSpec((4, d), lambda i: (0, 0)),
            pl.BlockSpec((d, d), lambda i: (0, 0)),
            pl.BlockSpec((1, d), lambda i: (0, 0)),
            pl.BlockSpec((d, d), lambda i: (0, 0)),
        ],
        out_specs=pl.BlockSpec((bn, d), lambda i: (i, 0)),
        out_shape=jax.ShapeDtypeStruct((n, d), jnp.float32),
    )(cnt3, cnt3, nt2, node_table, wl1t, bl1.reshape(1, d), wr1t)


def _layer2_head_tc(agg3, cnt3, x1, wl2t, bl2, wr2t, h1t, hb1, h2t, hb2, n, d, c):
    bn = 2000

    def body(p0_ref, p1_ref, c0_ref, c1_ref, x1_ref, wl_ref, bl_ref, wr_ref,
             h1_ref, hb1_ref, h2_ref, hb2_ref, o_ref):
        c4 = c0_ref[0] + c1_ref[0]
        deg = jnp.sum(c4, axis=1, keepdims=True)
        inv = 1.0 / jnp.maximum(deg, 1.0)
        agg = (p0_ref[0] + p1_ref[0]) * inv
        x1b = x1_ref[...]
        x2 = jnp.maximum(
            _dott(agg, wl_ref[...]) + bl_ref[...] + _dott(x1b, wr_ref[...]),
            0.0,
        )
        h = jnp.maximum(_dott(x2, h1_ref[...]) + hb1_ref[...], 0.0)
        o_ref[...] = _dott(h, h2_ref[...]) + hb2_ref[...]

    return pl.pallas_call(
        body,
        grid=(n // bn,),
        in_specs=[
            pl.BlockSpec((1, bn, d), lambda i: (0, i, 0)),
            pl.BlockSpec((1, bn, d), lambda i: (1, i, 0)),
            pl.BlockSpec((1, bn, 4), lambda i: (0, i, 0)),
            pl.BlockSpec((1, bn, 4), lambda i: (1, i, 0)),
            pl.BlockSpec((bn, d), lambda i: (i, 0)),
            pl.BlockSpec((d, d), lambda i: (0, 0)),
            pl.BlockSpec((1, d), lambda i: (0, 0)),
            pl.BlockSpec((d, d), lambda i: (0, 0)),
            pl.BlockSpec((d, d), lambda i: (0, 0)),
            pl.BlockSpec((1, d), lambda i: (0, 0)),
            pl.BlockSpec((c, d), lambda i: (0, 0)),
            pl.BlockSpec((1, c), lambda i: (0, 0)),
        ],
        out_specs=pl.BlockSpec((bn, c), lambda i: (i, 0)),
        out_shape=jax.ShapeDtypeStruct((n, c), jnp.float32),
    )(agg3, agg3, cnt3, cnt3, x1, wl2t, bl2.reshape(1, d), wr2t, h1t,
      hb1.reshape(1, d), h2t, hb2.reshape(1, c))


def kernel(node_type, edge_type, edge_index, y, node_table, edge_table,
           Wl1, bl1, Wr1, Wl2, bl2, Wr2, H1, hb1, H2, hb2):
    n = node_type.shape[0]
    e = edge_index.shape[1]
    d = node_table.shape[1]
    c = H2.shape[0]
    nw = e // W
    nw2 = e // WC

    src3 = edge_index[0].reshape(nw, 1, W)
    dst3 = edge_index[1].reshape(nw, 1, W)
    src2 = edge_index[0].reshape(nw2, WC)
    dst2 = edge_index[1].reshape(nw2, WC)
    z1d = jnp.zeros((4 * NPAD // 16,), jnp.float32)
    z128 = jnp.zeros((NPAD // 16, d), jnp.float32)

    cnt3 = _counts_sc(node_type, src2, dst2, z1d).reshape(2, NPAD, 4)
    nt2 = node_type.reshape(n, 1)
    x1 = _layer1_tc(cnt3, nt2, node_table, Wl1, bl1, Wr1, n, d)
    agg3 = _segsum_sc(x1, src3, dst3, z128)
    pred = _layer2_head_tc(agg3, cnt3, x1,
                           Wl2, bl2, Wr2, H1, hb1, H2, hb2, n, d, c)
    true_class = y[:, 1].astype(jnp.int32)
    return (pred, true_class, y)
